# baseline (speedup 1.0000x reference)
import functools

import jax
import jax.numpy as jnp
from jax import lax
from jax.experimental import pallas as pl
from jax.experimental.pallas import tpu as pltpu
from jax.experimental.pallas import tpu_sc as plsc

_ROWS = 2
_NBUF = 2


@functools.lru_cache(maxsize=None)
def _build(batch, seq, vocab, dim, n_workers):
    rows_per_w = batch // n_workers
    n_chunks = rows_per_w // _ROWS
    n_outer = n_chunks // _NBUF
    sub = max(d for d in range(8, 129, 8) if seq % d == 0)
    n_sub = seq // sub
    mesh = plsc.VectorSubcoreMesh(core_axis_name="c", subcore_axis_name="s")
    info = plsc.get_sparse_core_info()
    num_cores = info.num_cores

    @functools.partial(
        pl.kernel,
        mesh=mesh,
        compiler_params=pltpu.CompilerParams(use_tc_tiling_on_sc=False),
        out_type=jax.ShapeDtypeStruct((batch, seq, 128), jnp.float32),
        scratch_types=[
            pltpu.VMEM((_NBUF, _ROWS, seq), jnp.int32),
            pltpu.VMEM((_NBUF, _ROWS, seq, dim), jnp.float32),
            pltpu.VMEM((seq, dim), jnp.float32),
            pltpu.SemaphoreType.DMA,
            pltpu.SemaphoreType.DMA,
            pltpu.SemaphoreType.DMA,
            pltpu.SemaphoreType.DMA,
        ],
    )
    def emb(table_hbm, idx_hbm, pos_hbm, out_hbm, idx_v, rows_v, pos_v,
            gsem0, gsem1, ssem0, ssem1):
        gsem = (gsem0, gsem1)
        ssem = (ssem0, ssem1)
        wid = lax.axis_index("s") * num_cores + lax.axis_index("c")
        row_base = wid * rows_per_w
        pltpu.sync_copy(pos_hbm, pos_v)

        def out_slice(chunk_id, i):
            return out_hbm.at[row_base + chunk_id * _ROWS + i, :,
                              pl.ds(0, dim)]

        def stage(chunk_id, b):
            pltpu.sync_copy(
                idx_hbm.at[pl.ds(row_base + chunk_id * _ROWS, _ROWS)],
                idx_v.at[b])

            @plsc.parallel_loop(0, seq, 1, unroll=8)
            def _prefill(r):
                for i in range(_ROWS):
                    for cc in range(dim // 16):
                        sl = pl.ds(cc * 16, 16)
                        rows_v[b, i, r, sl] = pos_v[r, sl]

            for i in range(_ROWS):
                for j in range(n_sub):
                    pltpu.async_copy(
                        table_hbm.at[idx_v.at[b, i, pl.ds(j * sub, sub)]],
                        rows_v.at[b, i, pl.ds(j * sub, sub)],
                        gsem[b],
                        add=True,
                    )

        def wait_gathers(chunk_id, b):
            pltpu.make_async_copy(
                out_hbm.at[pl.ds(row_base + chunk_id * _ROWS, _ROWS), :,
                           pl.ds(0, dim)],
                rows_v.at[b], gsem[b]).wait()

        def fire_scatter(chunk_id, b):
            for i in range(_ROWS):
                pltpu.async_copy(rows_v.at[b, i], out_slice(chunk_id, i),
                                 ssem[b])

        def wait_scatter(chunk_id, b):
            for i in range(_ROWS):
                pltpu.make_async_copy(rows_v.at[b, i], out_slice(chunk_id, i),
                                      ssem[b]).wait()

        def outer(g, carry):
            for b in range(_NBUF):
                cid = g * _NBUF + b
                @pl.when(g >= 1)
                def _():
                    wait_scatter(cid - _NBUF, b)

                stage(cid, b)

                o = 1 - b
                pcid = cid - 1

                @pl.when(cid >= 1)
                def _():
                    wait_gathers(pcid, o)
                    fire_scatter(pcid, o)

            return carry

        lax.fori_loop(0, n_outer, outer, 0)

        last = n_chunks - 1
        bl = last % _NBUF
        wait_gathers(last, bl)
        fire_scatter(last, bl)
        wait_scatter(last - 1, 1 - bl)
        wait_scatter(last, bl)

    return emb


def kernel(input_ids, table, pos_encoding):
    b, s = input_ids.shape
    v, d = table.shape
    emb = _build(b, s, v, d, 32)
    out = emb(table, input_ids.astype(jnp.int32), pos_encoding[:s])
    return out[:, :, :d]


# speedup vs baseline: 1.0013x; 1.0013x over previous
import functools

import jax
import jax.numpy as jnp
from jax import lax
from jax.experimental import pallas as pl
from jax.experimental.pallas import tpu as pltpu
from jax.experimental.pallas import tpu_sc as plsc

_ROWS = 4
_NBUF = 2


@functools.lru_cache(maxsize=None)
def _build(batch, seq, vocab, dim, n_workers):
    rows_per_w = batch // n_workers
    n_chunks = rows_per_w // _ROWS
    n_outer = n_chunks // _NBUF
    sub = max(d for d in range(8, 129, 8) if seq % d == 0)
    n_sub = seq // sub
    mesh = plsc.VectorSubcoreMesh(core_axis_name="c", subcore_axis_name="s")
    info = plsc.get_sparse_core_info()
    num_cores = info.num_cores

    @functools.partial(
        pl.kernel,
        mesh=mesh,
        compiler_params=pltpu.CompilerParams(use_tc_tiling_on_sc=False),
        out_type=jax.ShapeDtypeStruct((batch, seq, 128), jnp.float32),
        scratch_types=[
            pltpu.VMEM((_NBUF, _ROWS, seq), jnp.int32),
            pltpu.VMEM((_NBUF, _ROWS, seq, dim), jnp.float32),
            pltpu.VMEM((seq, dim), jnp.float32),
            pltpu.SemaphoreType.DMA,
            pltpu.SemaphoreType.DMA,
            pltpu.SemaphoreType.DMA,
            pltpu.SemaphoreType.DMA,
        ],
    )
    def emb(table_hbm, idx_hbm, pos_hbm, out_hbm, idx_v, rows_v, pos_v,
            gsem0, gsem1, ssem0, ssem1):
        gsem = (gsem0, gsem1)
        ssem = (ssem0, ssem1)
        wid = lax.axis_index("s") * num_cores + lax.axis_index("c")
        row_base = wid * rows_per_w
        pltpu.sync_copy(pos_hbm, pos_v)

        def out_slice(chunk_id, i):
            return out_hbm.at[row_base + chunk_id * _ROWS + i, :,
                              pl.ds(0, dim)]

        def stage(chunk_id, b):
            pltpu.sync_copy(
                idx_hbm.at[pl.ds(row_base + chunk_id * _ROWS, _ROWS)],
                idx_v.at[b])

            @plsc.parallel_loop(0, seq, 1, unroll=8)
            def _prefill(r):
                for i in range(_ROWS):
                    for cc in range(dim // 16):
                        sl = pl.ds(cc * 16, 16)
                        rows_v[b, i, r, sl] = pos_v[r, sl]

            for i in range(_ROWS):
                for j in range(n_sub):
                    pltpu.async_copy(
                        table_hbm.at[idx_v.at[b, i, pl.ds(j * sub, sub)]],
                        rows_v.at[b, i, pl.ds(j * sub, sub)],
                        gsem[b],
                        add=True,
                    )

        def wait_gathers(chunk_id, b):
            pltpu.make_async_copy(
                out_hbm.at[pl.ds(row_base + chunk_id * _ROWS, _ROWS), :,
                           pl.ds(0, dim)],
                rows_v.at[b], gsem[b]).wait()

        def fire_scatter(chunk_id, b):
            for i in range(_ROWS):
                pltpu.async_copy(rows_v.at[b, i], out_slice(chunk_id, i),
                                 ssem[b])

        def wait_scatter(chunk_id, b):
            for i in range(_ROWS):
                pltpu.make_async_copy(rows_v.at[b, i], out_slice(chunk_id, i),
                                      ssem[b]).wait()

        def outer(g, carry):
            for b in range(_NBUF):
                cid = g * _NBUF + b
                @pl.when(g >= 1)
                def _():
                    wait_scatter(cid - _NBUF, b)

                stage(cid, b)

                o = 1 - b
                pcid = cid - 1

                @pl.when(cid >= 1)
                def _():
                    wait_gathers(pcid, o)
                    fire_scatter(pcid, o)

            return carry

        lax.fori_loop(0, n_outer, outer, 0)

        last = n_chunks - 1
        bl = last % _NBUF
        wait_gathers(last, bl)
        fire_scatter(last, bl)
        wait_scatter(last - 1, 1 - bl)
        wait_scatter(last, bl)

    return emb


def kernel(input_ids, table, pos_encoding):
    b, s = input_ids.shape
    v, d = table.shape
    emb = _build(b, s, v, d, 32)
    out = emb(table, input_ids.astype(jnp.int32), pos_encoding[:s])
    return out[:, :, :d]
